# baseline (speedup 1.0000x reference)
import functools

import jax
import jax.numpy as jnp
from jax import lax
from jax.experimental import pallas as pl
from jax.experimental.pallas import tpu as pltpu
from jax.experimental.pallas import tpu_sc as plsc

N_NODES = 50000
EMBED = 256
COMP = 128
T = 8
PER_T = 10000

NC = 2
NS = 16
PER_TILE = PER_T // NS
PER_TILE_PAD = 640
TILE_STRIDE = 624
N_STREAMS = PER_TILE_PAD // 128
N_CHUNKS = 3
CHUNK = 8448
ROWS_PER_TILE = CHUNK // NS
DUMMY_ROW = CHUNK
SC1_BASE = N_NODES - N_CHUNKS * CHUNK


def _pool_block(x_ref, o_ref):
    r = lax.broadcasted_iota(jnp.int32, (EMBED, COMP), 0)
    c = lax.broadcasted_iota(jnp.int32, (EMBED, COMP), 1)
    p = jnp.where(r // 2 == c, jnp.float32(0.5), jnp.float32(0.0))
    o_ref[...] = lax.dot(
        x_ref[...], p,
        precision=lax.Precision.HIGHEST,
        preferred_element_type=jnp.float32,
    )


def _pool(x, block_rows):
    rows = x.shape[0]
    return pl.pallas_call(
        _pool_block,
        grid=(rows // block_rows,),
        in_specs=[pl.BlockSpec((block_rows, EMBED), lambda i: (i, 0))],
        out_specs=pl.BlockSpec((block_rows, COMP), lambda i: (i, 0)),
        out_shape=jax.ShapeDtypeStruct((rows, COMP), jnp.float32),
    )(x)


def _sc_body(px_hbm, pe_hbm, idx_hbm, out_hbm, staging, idx_v, remap, srcpos,
             pepos, table, sem_g0, sem_g1, sem_w):
    c = lax.axis_index("c")
    s = lax.axis_index("s")
    gsems = [sem_g0, sem_g1]

    def pingpong_streams(src_of, dst_of):
        gathers = [None] * N_STREAMS
        gathers[0] = pltpu.async_copy(src_of(0), staging.at[0], gsems[0])
        for j in range(N_STREAMS):
            if j + 1 < N_STREAMS:
                gathers[j + 1] = pltpu.async_copy(
                    src_of(j + 1), staging.at[(j + 1) % 2],
                    gsems[(j + 1) % 2])
            gathers[j].wait()
            pltpu.sync_copy(staging.at[j % 2], dst_of(j), **dst_of.kw)

    for k in range(N_CHUNKS):
        base = c * SC1_BASE + k * CHUNK

        pltpu.sync_copy(
            pe_hbm.at[pl.ds(base + s * ROWS_PER_TILE, ROWS_PER_TILE)],
            table.at[pl.ds(s * ROWS_PER_TILE, ROWS_PER_TILE)])
        plsc.subcore_barrier()

        def per_t(t, _):
            pltpu.sync_copy(
                idx_hbm.at[pl.ds((t * NS + s) * PER_TILE_PAD, PER_TILE_PAD)],
                idx_v)
            src0 = t * PER_T + s * TILE_STRIDE

            lanes = lax.iota(jnp.int32, 16)
            for i in range(PER_TILE_PAD // 16):
                v = idx_v[pl.ds(i * 16, 16)]
                local = v - base
                ok = (local >= 0) & (local < CHUNK)
                remap[i // 8, pl.ds((i % 8) * 16, 16)] = jnp.where(
                    ok, local, DUMMY_ROW)
                srcpos[i // 8, pl.ds((i % 8) * 16, 16)] = jnp.where(
                    ok, src0 + i * 16 + lanes, -1)
                pepos[i // 8, pl.ds((i % 8) * 16, 16)] = jnp.where(
                    ok, v, -1)

            def add_src(j):
                return px_hbm.at[plsc.Indices(srcpos.at[j],
                                              ignored_value=-1)]

            def add_dst(j):
                return table.at[plsc.Indices(remap.at[j],
                                             ignored_value=DUMMY_ROW)]
            add_dst.kw = dict(add=True)
            pingpong_streams(add_src, add_dst)

            plsc.subcore_barrier()

            wo = pltpu.async_copy(
                table.at[pl.ds(s * ROWS_PER_TILE, ROWS_PER_TILE)],
                out_hbm.at[t, pl.ds(base + s * ROWS_PER_TILE,
                                    ROWS_PER_TILE)],
                sem_w)

            @pl.when(t < T - 1)
            def _():
                def rep_src(j):
                    return pe_hbm.at[plsc.Indices(pepos.at[j],
                                                  ignored_value=-1)]

                def rep_dst(j):
                    return table.at[plsc.Indices(remap.at[j],
                                                 ignored_value=DUMMY_ROW)]
                rep_dst.kw = dict()
                wo.wait()
                plsc.subcore_barrier()
                pingpong_streams(rep_src, rep_dst)
                plsc.subcore_barrier()

            @pl.when(t == T - 1)
            def _():
                wo.wait()
                plsc.subcore_barrier()
            return 0

        lax.fori_loop(0, T, per_t, 0)


@jax.jit
def kernel(x, entire_x, indices):
    px = _pool(x, 5000)
    pe = _pool(entire_x, 5000)

    j = jnp.arange(PER_TILE_PAD)[None, :]
    srow = jnp.arange(NS)[:, None]
    pos = TILE_STRIDE * srow + j
    valid = (j >= srow) & (j < srow + PER_TILE)
    gathered = jnp.take(indices.astype(jnp.int32), pos, axis=1)
    idx3 = jnp.where(valid[None], gathered, N_NODES).reshape(-1)

    mesh = plsc.VectorSubcoreMesh(core_axis_name="c", subcore_axis_name="s")
    sc = pl.kernel(
        _sc_body,
        out_type=jax.ShapeDtypeStruct((T, N_NODES, COMP), jnp.float32),
        mesh=mesh,
        scratch_types=[
            pltpu.VMEM((2, 128, COMP), jnp.float32),
            pltpu.VMEM((PER_TILE_PAD,), jnp.int32),
            pltpu.VMEM((N_STREAMS, 128), jnp.int32),
            pltpu.VMEM((N_STREAMS, 128), jnp.int32),
            pltpu.VMEM((N_STREAMS, 128), jnp.int32),
            pltpu.VMEM_SHARED((CHUNK, COMP), jnp.float32),
            pltpu.SemaphoreType.DMA,
            pltpu.SemaphoreType.DMA,
            pltpu.SemaphoreType.DMA,
        ],
    )
    return sc(px, pe, idx3)


# speedup vs baseline: 1.2643x; 1.2643x over previous
import functools

import jax
import jax.numpy as jnp
from jax import lax
from jax.experimental import pallas as pl
from jax.experimental.pallas import tpu as pltpu
from jax.experimental.pallas import tpu_sc as plsc

N_NODES = 50000
EMBED = 256
COMP = 128
T = 8
PER_T = 10000

NC = 2
NS = 16
PER_TILE = PER_T // NS
PER_TILE_PAD = 640
TILE_STRIDE = 624
N_STREAMS = PER_TILE_PAD // 128
N_CHUNKS = 3
CHUNK = 8448
ROWS_PER_TILE = CHUNK // NS
DUMMY_ROW = CHUNK
SC1_BASE = N_NODES - N_CHUNKS * CHUNK


XBLK = 3200
EBLK = 2000


def _pool_mat():
    r = lax.broadcasted_iota(jnp.int32, (EMBED, COMP), 0)
    c = lax.broadcasted_iota(jnp.int32, (EMBED, COMP), 1)
    return jnp.where(r // 2 == c, jnp.float32(0.5), jnp.float32(0.0))


def _pool_block(x_ref, e_ref, px_ref, pe_ref):
    p = _pool_mat()
    px_ref[...] = lax.dot(x_ref[...], p,
                          precision=lax.Precision.HIGHEST,
                          preferred_element_type=jnp.float32)
    pe_ref[...] = lax.dot(e_ref[...], p,
                          precision=lax.Precision.HIGHEST,
                          preferred_element_type=jnp.float32)


def _pool_both(x, entire_x):
    return pl.pallas_call(
        _pool_block,
        grid=(x.shape[0] // XBLK,),
        in_specs=[pl.BlockSpec((XBLK, EMBED), lambda i: (i, 0)),
                  pl.BlockSpec((EBLK, EMBED), lambda i: (i, 0))],
        out_specs=[pl.BlockSpec((XBLK, COMP), lambda i: (i, 0)),
                   pl.BlockSpec((EBLK, COMP), lambda i: (i, 0))],
        out_shape=[jax.ShapeDtypeStruct((x.shape[0], COMP), jnp.float32),
                   jax.ShapeDtypeStruct((entire_x.shape[0], COMP),
                                        jnp.float32)],
    )(x, entire_x)


def _sc_body(px_hbm, pe_hbm, idx_hbm, out_hbm, staging, idx_v, remap, srcpos,
             table, sem_init, sem_g0, sem_g1):
    c = lax.axis_index("c")
    s = lax.axis_index("s")
    gsems = [sem_g0, sem_g1]

    def per_t(t, _):
        pltpu.sync_copy(
            idx_hbm.at[pl.ds((t * NS + s) * PER_TILE_PAD, PER_TILE_PAD)],
            idx_v)
        src0 = t * PER_T + s * TILE_STRIDE

        for k in range(N_CHUNKS):
            base = c * SC1_BASE + k * CHUNK

            init = pltpu.async_copy(
                pe_hbm.at[pl.ds(base + s * ROWS_PER_TILE, ROWS_PER_TILE)],
                table.at[pl.ds(s * ROWS_PER_TILE, ROWS_PER_TILE)],
                sem_init)

            lanes = lax.iota(jnp.int32, 16)
            for i in range(PER_TILE_PAD // 16):
                v = idx_v[pl.ds(i * 16, 16)]
                local = v - base
                ok = (local >= 0) & (local < CHUNK)
                remap[i // 8, pl.ds((i % 8) * 16, 16)] = jnp.where(
                    ok, local, DUMMY_ROW)
                srcpos[i // 8, pl.ds((i % 8) * 16, 16)] = jnp.where(
                    ok, src0 + i * 16 + lanes, -1)

            gathers = [None] * N_STREAMS
            gathers[0] = pltpu.async_copy(
                px_hbm.at[plsc.Indices(srcpos.at[0], ignored_value=-1)],
                staging.at[0], gsems[0])

            init.wait()
            plsc.subcore_barrier()

            for j in range(N_STREAMS):
                if j + 1 < N_STREAMS:
                    gathers[j + 1] = pltpu.async_copy(
                        px_hbm.at[plsc.Indices(srcpos.at[j + 1],
                                               ignored_value=-1)],
                        staging.at[(j + 1) % 2], gsems[(j + 1) % 2])
                gathers[j].wait()
                pltpu.sync_copy(
                    staging.at[j % 2],
                    table.at[plsc.Indices(remap.at[j],
                                          ignored_value=DUMMY_ROW)],
                    add=True)

            plsc.subcore_barrier()

            pltpu.sync_copy(
                table.at[pl.ds(s * ROWS_PER_TILE, ROWS_PER_TILE)],
                out_hbm.at[t, pl.ds(base + s * ROWS_PER_TILE, ROWS_PER_TILE)])
        return 0

    lax.fori_loop(0, T, per_t, 0)


@jax.jit
def kernel(x, entire_x, indices):
    px, pe = _pool_both(x, entire_x)

    j = jnp.arange(PER_TILE_PAD)[None, :]
    srow = jnp.arange(NS)[:, None]
    pos = TILE_STRIDE * srow + j
    valid = (j >= srow) & (j < srow + PER_TILE)
    gathered = jnp.take(indices.astype(jnp.int32), pos, axis=1)
    idx3 = jnp.where(valid[None], gathered, N_NODES).reshape(-1)

    mesh = plsc.VectorSubcoreMesh(core_axis_name="c", subcore_axis_name="s")
    sc = pl.kernel(
        _sc_body,
        out_type=jax.ShapeDtypeStruct((T, N_NODES, COMP), jnp.float32),
        mesh=mesh,
        scratch_types=[
            pltpu.VMEM((2, 128, COMP), jnp.float32),
            pltpu.VMEM((PER_TILE_PAD,), jnp.int32),
            pltpu.VMEM((N_STREAMS, 128), jnp.int32),
            pltpu.VMEM((N_STREAMS, 128), jnp.int32),
            pltpu.VMEM_SHARED((CHUNK, COMP), jnp.float32),
            pltpu.SemaphoreType.DMA,
            pltpu.SemaphoreType.DMA,
            pltpu.SemaphoreType.DMA,
        ],
    )
    return sc(px, pe, idx3)


# speedup vs baseline: 1.3076x; 1.0343x over previous
import functools

import jax
import jax.numpy as jnp
from jax import lax
from jax.experimental import pallas as pl
from jax.experimental.pallas import tpu as pltpu
from jax.experimental.pallas import tpu_sc as plsc

N_NODES = 50000
EMBED = 256
COMP = 128
T = 8
PER_T = 10000

NC = 2
NS = 16
PER_TILE = PER_T // NS
PER_TILE_PAD = 640
TILE_STRIDE = 624
N_STREAMS = PER_TILE_PAD // 128
N_CHUNKS = 3
CHUNK = 8448
ROWS_PER_TILE = CHUNK // NS
DUMMY_ROW = CHUNK
SC1_BASE = N_NODES - N_CHUNKS * CHUNK


HALF_X = T // 2 * PER_T
XBLK = 1600
EBLK = 2000
XBLK2 = 4000


def _pool_mat():
    r = lax.broadcasted_iota(jnp.int32, (EMBED, COMP), 0)
    c = lax.broadcasted_iota(jnp.int32, (EMBED, COMP), 1)
    return jnp.where(r // 2 == c, jnp.float32(0.5), jnp.float32(0.0))


def _pool_block2(x_ref, e_ref, px_ref, pe_ref):
    p = _pool_mat()
    px_ref[...] = lax.dot(x_ref[...], p,
                          precision=lax.Precision.HIGHEST,
                          preferred_element_type=jnp.float32)
    pe_ref[...] = lax.dot(e_ref[...], p,
                          precision=lax.Precision.HIGHEST,
                          preferred_element_type=jnp.float32)


def _pool_block1(x_ref, px_ref):
    px_ref[...] = lax.dot(x_ref[...], _pool_mat(),
                          precision=lax.Precision.HIGHEST,
                          preferred_element_type=jnp.float32)


def _pool_first(x, entire_x):
    return pl.pallas_call(
        _pool_block2,
        grid=(HALF_X // XBLK,),
        in_specs=[pl.BlockSpec((XBLK, EMBED), lambda i: (i, 0)),
                  pl.BlockSpec((EBLK, EMBED), lambda i: (i, 0))],
        out_specs=[pl.BlockSpec((XBLK, COMP), lambda i: (i, 0)),
                   pl.BlockSpec((EBLK, COMP), lambda i: (i, 0))],
        out_shape=[jax.ShapeDtypeStruct((HALF_X, COMP), jnp.float32),
                   jax.ShapeDtypeStruct((entire_x.shape[0], COMP),
                                        jnp.float32)],
    )(x, entire_x)


def _pool_second(x):
    nblk = HALF_X // XBLK2
    return pl.pallas_call(
        _pool_block1,
        grid=(nblk,),
        in_specs=[pl.BlockSpec((XBLK2, EMBED), lambda i: (i + nblk, 0))],
        out_specs=pl.BlockSpec((XBLK2, COMP), lambda i: (i, 0)),
        out_shape=jax.ShapeDtypeStruct((HALF_X, COMP), jnp.float32),
    )(x)


def _make_sc_body(t0, t1):
    def _sc_body(px_hbm, pe_hbm, idx_hbm, out_hbm, staging, idx_v, remap,
                 srcpos, table, sem_init, sem_g0, sem_g1):
        c = lax.axis_index("c")
        s = lax.axis_index("s")
        gsems = [sem_g0, sem_g1]

        def per_t(t, _):
            pltpu.sync_copy(
                idx_hbm.at[pl.ds((t * NS + s) * PER_TILE_PAD, PER_TILE_PAD)],
                idx_v)
            src0 = (t - t0) * PER_T + s * TILE_STRIDE

            for k in range(N_CHUNKS):
                base = c * SC1_BASE + k * CHUNK

                init = pltpu.async_copy(
                    pe_hbm.at[pl.ds(base + s * ROWS_PER_TILE,
                                    ROWS_PER_TILE)],
                    table.at[pl.ds(s * ROWS_PER_TILE, ROWS_PER_TILE)],
                    sem_init)

                lanes = lax.iota(jnp.int32, 16)
                for i in range(PER_TILE_PAD // 16):
                    v = idx_v[pl.ds(i * 16, 16)]
                    local = v - base
                    ok = (local >= 0) & (local < CHUNK)
                    remap[i // 8, pl.ds((i % 8) * 16, 16)] = jnp.where(
                        ok, local, DUMMY_ROW)
                    srcpos[i // 8, pl.ds((i % 8) * 16, 16)] = jnp.where(
                        ok, src0 + i * 16 + lanes, -1)

                gathers = [None] * N_STREAMS
                gathers[0] = pltpu.async_copy(
                    px_hbm.at[plsc.Indices(srcpos.at[0], ignored_value=-1)],
                    staging.at[0], gsems[0])

                init.wait()
                plsc.subcore_barrier()

                for j in range(N_STREAMS):
                    if j + 1 < N_STREAMS:
                        gathers[j + 1] = pltpu.async_copy(
                            px_hbm.at[plsc.Indices(srcpos.at[j + 1],
                                                   ignored_value=-1)],
                            staging.at[(j + 1) % 2], gsems[(j + 1) % 2])
                    gathers[j].wait()
                    pltpu.sync_copy(
                        staging.at[j % 2],
                        table.at[plsc.Indices(remap.at[j],
                                              ignored_value=DUMMY_ROW)],
                        add=True)

                plsc.subcore_barrier()

                pltpu.sync_copy(
                    table.at[pl.ds(s * ROWS_PER_TILE, ROWS_PER_TILE)],
                    out_hbm.at[t, pl.ds(base + s * ROWS_PER_TILE,
                                        ROWS_PER_TILE)])
            return 0

        lax.fori_loop(t0, t1, per_t, 0)

    return _sc_body


@jax.jit
def kernel(x, entire_x, indices):
    px0, pe = _pool_first(x, entire_x)

    j = jnp.arange(PER_TILE_PAD)[None, :]
    srow = jnp.arange(NS)[:, None]
    pos = TILE_STRIDE * srow + j
    valid = (j >= srow) & (j < srow + PER_TILE)
    gathered = jnp.take(indices.astype(jnp.int32), pos, axis=1)
    idx3 = jnp.where(valid[None], gathered, N_NODES).reshape(-1)

    mesh = plsc.VectorSubcoreMesh(core_axis_name="c", subcore_axis_name="s")
    scratch = [
        pltpu.VMEM((2, 128, COMP), jnp.float32),
        pltpu.VMEM((PER_TILE_PAD,), jnp.int32),
        pltpu.VMEM((N_STREAMS, 128), jnp.int32),
        pltpu.VMEM((N_STREAMS, 128), jnp.int32),
        pltpu.VMEM_SHARED((CHUNK, COMP), jnp.float32),
        pltpu.SemaphoreType.DMA,
        pltpu.SemaphoreType.DMA,
        pltpu.SemaphoreType.DMA,
    ]

    sc1 = pl.kernel(
        _make_sc_body(0, T // 2),
        out_type=jax.ShapeDtypeStruct((T, N_NODES, COMP), jnp.float32),
        mesh=mesh,
        scratch_types=scratch,
    )
    px1 = _pool_second(x)
    out0 = sc1(px0, pe, idx3)

    out_ref = jax.new_ref(out0)
    sc2 = pl.kernel(
        _make_sc_body(T // 2, T),
        out_type=(),
        mesh=mesh,
        scratch_types=scratch,
    )
    sc2(px1, pe, idx3, out_ref)
    return out_ref[...]
